# baseline (speedup 1.0000x reference)
import functools

import jax
import jax.numpy as jnp
from jax import lax
from jax.experimental import pallas as pl
from jax.experimental.pallas import tpu as pltpu
from jax.experimental.pallas import tpu_sc as plsc

B = 16384
DIM = 64
NCTX = 5
NC = 2
NS = 16
NW = NC * NS
BPW = B // NW
CH = 64
NCHUNK = BPW // CH
LANES = 16
NG = CH // LANES
CIB = 3
SPLIT = 499968
PH = 500032


def _tc_pack(table):
    t_t = table.T

    def body(a_ref, b_ref, o_ref):
        o_ref[:, 0:DIM] = a_ref[...].T
        o_ref[:, DIM:128] = b_ref[...].T

    nblk = (PH + 127) // 128
    return pl.pallas_call(
        body,
        grid=(nblk,),
        in_specs=[
            pl.BlockSpec((DIM, 128), lambda i: (0, i)),
            pl.BlockSpec((DIM, 128), lambda i: (0, SPLIT // 128 + i)),
        ],
        out_specs=pl.BlockSpec((128, 128), lambda i: (i, 0)),
        out_shape=jax.ShapeDtypeStruct((PH, 128), jnp.float32),
    )(t_t, t_t)


def _make_kernel():
    mesh = plsc.VectorSubcoreMesh(core_axis_name="c", subcore_axis_name="s")

    @functools.partial(
        pl.kernel,
        out_type=jax.ShapeDtypeStruct((B * NCTX,), jnp.float32),
        mesh=mesh,
        compiler_params=pltpu.CompilerParams(needs_layout_passes=False),
        scratch_types=[
            pltpu.VMEM((1, CH), jnp.int32),
            pltpu.VMEM((CIB, 128), jnp.int32),
            pltpu.VMEM((1, CH), jnp.int32),
            pltpu.VMEM((CIB, 128), jnp.int32),
            pltpu.VMEM((CH, 128), jnp.float32),
            pltpu.VMEM((CH * NCTX, 128), jnp.float32),
            pltpu.VMEM((CH * NCTX,), jnp.float32),
            pltpu.SemaphoreType.DMA,
            pltpu.SemaphoreType.DMA,
        ],
    )
    def body(tgt_hbm, ctx_hbm, ttab_hbm, ctab_hbm, out_hbm,
             traw, craw, tidx, cidx, tgt_sl, ctx_sl, out_v, sem, sem2):
        wid = lax.axis_index("s") * NC + lax.axis_index("c")
        lane = lax.iota(jnp.int32, LANES)

        @pl.loop(0, NCHUNK)
        def _chunk(ch):
            base = (wid * NCHUNK + ch) * CH
            cb = base * NCTX
            icps = [pltpu.async_copy(tgt_hbm.at[pl.ds(base, CH)],
                                     traw.at[0], sem2),
                    pltpu.async_copy(ctx_hbm.at[pl.ds(cb, 128)],
                                     craw.at[0], sem2),
                    pltpu.async_copy(ctx_hbm.at[pl.ds(cb + 128, 128)],
                                     craw.at[1], sem2),
                    pltpu.async_copy(ctx_hbm.at[pl.ds(cb + 256, 64)],
                                     craw.at[2, pl.ds(0, 64)], sem2)]
            for cp in icps:
                cp.wait()
            for v in range(CH // LANES):
                r = traw[0, pl.ds(v * LANES, LANES)]
                tidx[0, pl.ds(v * LANES, LANES)] = jnp.where(
                    r >= SPLIT, r - SPLIT, r)
            for j in range(CIB):
                n = 128 if j < 2 else 64
                for v in range(n // LANES):
                    r = craw[j, pl.ds(v * LANES, LANES)]
                    cidx[j, pl.ds(v * LANES, LANES)] = jnp.where(
                        r >= SPLIT, r - SPLIT, r)
            cps = [pltpu.async_copy(ttab_hbm.at[tidx.at[0]], tgt_sl, sem),
                   pltpu.async_copy(ctab_hbm.at[cidx.at[0]],
                                    ctx_sl.at[pl.ds(0, 128)], sem),
                   pltpu.async_copy(ctab_hbm.at[cidx.at[1]],
                                    ctx_sl.at[pl.ds(128, 128)], sem),
                   pltpu.async_copy(ctab_hbm.at[cidx.at[2, pl.ds(0, 64)]],
                                    ctx_sl.at[pl.ds(256, 64)], sem)]
            for cp in cps:
                cp.wait()

            @pl.loop(0, NG)
            def _grp(g):
                trow = g * LANES + lane
                tr = plsc.load_gather(traw.at[0], [trow])
                tc0 = jnp.where(tr >= SPLIT, DIM, 0)
                pvecs, cc0, accs = [], [], []
                for c in range(NCTX):
                    p = trow * NCTX + c
                    cr = plsc.load_gather(craw, [p >> 7, p & 127])
                    pvecs.append(p)
                    cc0.append(jnp.where(cr >= SPLIT, DIM, 0))
                    accs.append(jnp.zeros((LANES,), jnp.float32))
                for d in range(DIM):
                    tv = plsc.load_gather(tgt_sl, [trow, tc0 + d])
                    for c in range(NCTX):
                        cv = plsc.load_gather(ctx_sl, [pvecs[c], cc0[c] + d])
                        accs[c] = accs[c] + cv * tv
                for c in range(NCTX):
                    plsc.store_scatter(out_v, [pvecs[c]], accs[c])

            pltpu.sync_copy(out_v, out_hbm.at[pl.ds(cb, CH * NCTX)])

    return body


_sc_kernel = _make_kernel()


def kernel(target, context, target_table, context_table):
    tgt1 = target.reshape(B).astype(jnp.int32)
    ctx1 = context.reshape(B * NCTX).astype(jnp.int32)
    ttab = _tc_pack(target_table)
    ctab = _tc_pack(context_table)
    flat = _sc_kernel(tgt1, ctx1, ttab, ctab)
    return flat.reshape(B, NCTX)


# speedup vs baseline: 5.5981x; 5.5981x over previous
import functools

import jax
import jax.numpy as jnp
from jax import lax
from jax.experimental import pallas as pl
from jax.experimental.pallas import tpu as pltpu
from jax.experimental.pallas import tpu_sc as plsc

B = 16384
DIM = 64
NCTX = 5
NC = 2
NS = 16
NW = NC * NS
BPW = B // NW
CH = 64
NCHUNK = BPW // CH
LANES = 16
NG = CH // LANES
CIB = 3
SPLIT = 499712
PH = 500288


def _tc_pack(table):
    t_t = table.T
    eye = jnp.eye(DIM, dtype=jnp.float32)
    bn = 2048

    def body(a_ref, b_ref, eye_ref, o_ref):
        e = eye_ref[...]
        dn = (((0,), (0,)), ((), ()))
        o_ref[:, 0:DIM] = lax.dot_general(
            a_ref[...], e, dn, preferred_element_type=jnp.float32)
        o_ref[:, DIM:128] = lax.dot_general(
            b_ref[...], e, dn, preferred_element_type=jnp.float32)

    nblk = (PH + bn - 1) // bn
    return pl.pallas_call(
        body,
        grid=(nblk,),
        in_specs=[
            pl.BlockSpec((DIM, bn), lambda i: (0, i)),
            pl.BlockSpec((DIM, bn), lambda i: (0, SPLIT // bn + i)),
            pl.BlockSpec((DIM, DIM), lambda i: (0, 0)),
        ],
        out_specs=pl.BlockSpec((bn, 128), lambda i: (i, 0)),
        out_shape=jax.ShapeDtypeStruct((PH, 128), jnp.float32),
    )(t_t, t_t, eye)


def _make_kernel():
    mesh = plsc.VectorSubcoreMesh(core_axis_name="c", subcore_axis_name="s")

    @functools.partial(
        pl.kernel,
        out_type=jax.ShapeDtypeStruct((B * NCTX,), jnp.float32),
        mesh=mesh,
        compiler_params=pltpu.CompilerParams(needs_layout_passes=False),
        scratch_types=[
            pltpu.VMEM((1, CH), jnp.int32),
            pltpu.VMEM((CIB, 128), jnp.int32),
            pltpu.VMEM((1, CH), jnp.int32),
            pltpu.VMEM((CIB, 128), jnp.int32),
            pltpu.VMEM((CH, 128), jnp.float32),
            pltpu.VMEM((CH * NCTX, 128), jnp.float32),
            pltpu.VMEM((CH * NCTX,), jnp.float32),
            pltpu.SemaphoreType.DMA,
            pltpu.SemaphoreType.DMA,
        ],
    )
    def body(tgt_hbm, ctx_hbm, ttab_hbm, ctab_hbm, out_hbm,
             traw, craw, tidx, cidx, tgt_sl, ctx_sl, out_v, sem, sem2):
        wid = lax.axis_index("s") * NC + lax.axis_index("c")
        lane = lax.iota(jnp.int32, LANES)

        @pl.loop(0, NCHUNK)
        def _chunk(ch):
            base = (wid * NCHUNK + ch) * CH
            cb = base * NCTX
            icps = [pltpu.async_copy(tgt_hbm.at[pl.ds(base, CH)],
                                     traw.at[0], sem2),
                    pltpu.async_copy(ctx_hbm.at[pl.ds(cb, 128)],
                                     craw.at[0], sem2),
                    pltpu.async_copy(ctx_hbm.at[pl.ds(cb + 128, 128)],
                                     craw.at[1], sem2),
                    pltpu.async_copy(ctx_hbm.at[pl.ds(cb + 256, 64)],
                                     craw.at[2, pl.ds(0, 64)], sem2)]
            for cp in icps:
                cp.wait()
            for v in range(CH // LANES):
                r = traw[0, pl.ds(v * LANES, LANES)]
                tidx[0, pl.ds(v * LANES, LANES)] = jnp.where(
                    r >= SPLIT, r - SPLIT, r)
            for j in range(CIB):
                n = 128 if j < 2 else 64
                for v in range(n // LANES):
                    r = craw[j, pl.ds(v * LANES, LANES)]
                    cidx[j, pl.ds(v * LANES, LANES)] = jnp.where(
                        r >= SPLIT, r - SPLIT, r)
            cps = [pltpu.async_copy(ttab_hbm.at[tidx.at[0]], tgt_sl, sem),
                   pltpu.async_copy(ctab_hbm.at[cidx.at[0]],
                                    ctx_sl.at[pl.ds(0, 128)], sem),
                   pltpu.async_copy(ctab_hbm.at[cidx.at[1]],
                                    ctx_sl.at[pl.ds(128, 128)], sem),
                   pltpu.async_copy(ctab_hbm.at[cidx.at[2, pl.ds(0, 64)]],
                                    ctx_sl.at[pl.ds(256, 64)], sem)]
            for cp in cps:
                cp.wait()

            @pl.loop(0, NG)
            def _grp(g):
                trow = g * LANES + lane
                tr = plsc.load_gather(traw.at[0], [trow])
                tc0 = jnp.where(tr >= SPLIT, DIM, 0)
                pvecs, cc0, accs = [], [], []
                for c in range(NCTX):
                    p = trow * NCTX + c
                    cr = plsc.load_gather(craw, [p >> 7, p & 127])
                    pvecs.append(p)
                    cc0.append(jnp.where(cr >= SPLIT, DIM, 0))
                    accs.append(jnp.zeros((LANES,), jnp.float32))
                for d in range(DIM):
                    tv = plsc.load_gather(tgt_sl, [trow, tc0 + d])
                    for c in range(NCTX):
                        cv = plsc.load_gather(ctx_sl, [pvecs[c], cc0[c] + d])
                        accs[c] = accs[c] + cv * tv
                for c in range(NCTX):
                    plsc.store_scatter(out_v, [pvecs[c]], accs[c])

            pltpu.sync_copy(out_v, out_hbm.at[pl.ds(cb, CH * NCTX)])

    return body


_sc_kernel = _make_kernel()


def kernel(target, context, target_table, context_table):
    tgt1 = target.reshape(B).astype(jnp.int32)
    ctx1 = context.reshape(B * NCTX).astype(jnp.int32)
    ttab = _tc_pack(target_table)
    ctab = _tc_pack(context_table)
    flat = _sc_kernel(tgt1, ctx1, ttab, ctab)
    return flat.reshape(B, NCTX)


# speedup vs baseline: 6.7262x; 1.2015x over previous
import functools

import jax
import jax.numpy as jnp
from jax import lax
from jax.experimental import pallas as pl
from jax.experimental.pallas import tpu as pltpu
from jax.experimental.pallas import tpu_sc as plsc

B = 16384
DIM = 64
NCTX = 5
NC = 2
NS = 16
NW = NC * NS
BPW = B // NW
CH = 64
NCHUNK = BPW // CH
LANES = 16
NG = CH // LANES
CIB = 3
SPLIT = 499712
PH = 500288


def _tc_pack(table):
    t_t = table.T
    eye = jnp.eye(DIM, dtype=jnp.float32)
    bn = 4096

    def body(a_ref, b_ref, eye_ref, o_ref):
        e = eye_ref[...]
        dn = (((0,), (0,)), ((), ()))
        o_ref[:, 0:DIM] = lax.dot_general(
            a_ref[...], e, dn, preferred_element_type=jnp.float32)
        o_ref[:, DIM:128] = lax.dot_general(
            b_ref[...], e, dn, preferred_element_type=jnp.float32)

    nblk = (PH + bn - 1) // bn
    return pl.pallas_call(
        body,
        grid=(nblk,),
        in_specs=[
            pl.BlockSpec((DIM, bn), lambda i: (0, i)),
            pl.BlockSpec((DIM, bn), lambda i: (0, SPLIT // bn + i)),
            pl.BlockSpec((DIM, DIM), lambda i: (0, 0)),
        ],
        out_specs=pl.BlockSpec((bn, 128), lambda i: (i, 0)),
        out_shape=jax.ShapeDtypeStruct((PH, 128), jnp.float32),
    )(t_t, t_t, eye)


def _make_kernel():
    mesh = plsc.VectorSubcoreMesh(core_axis_name="c", subcore_axis_name="s")

    @functools.partial(
        pl.kernel,
        out_type=jax.ShapeDtypeStruct((B * NCTX,), jnp.float32),
        mesh=mesh,
        compiler_params=pltpu.CompilerParams(needs_layout_passes=False),
        scratch_types=[
            pltpu.VMEM((1, CH), jnp.int32),
            pltpu.VMEM((CIB, 128), jnp.int32),
            pltpu.VMEM((1, CH), jnp.int32),
            pltpu.VMEM((CIB, 128), jnp.int32),
            pltpu.VMEM((CH, 128), jnp.float32),
            pltpu.VMEM((CH * NCTX, 128), jnp.float32),
            pltpu.VMEM((CH * NCTX,), jnp.float32),
            pltpu.SemaphoreType.DMA,
            pltpu.SemaphoreType.DMA,
        ],
    )
    def body(tgt_hbm, ctx_hbm, ttab_hbm, ctab_hbm, out_hbm,
             traw, craw, tidx, cidx, tgt_sl, ctx_sl, out_v, sem, sem2):
        wid = lax.axis_index("s") * NC + lax.axis_index("c")
        lane = lax.iota(jnp.int32, LANES)

        @pl.loop(0, NCHUNK)
        def _chunk(ch):
            base = (wid * NCHUNK + ch) * CH
            cb = base * NCTX
            icps = [pltpu.async_copy(tgt_hbm.at[pl.ds(base, CH)],
                                     traw.at[0], sem2),
                    pltpu.async_copy(ctx_hbm.at[pl.ds(cb, 128)],
                                     craw.at[0], sem2),
                    pltpu.async_copy(ctx_hbm.at[pl.ds(cb + 128, 128)],
                                     craw.at[1], sem2),
                    pltpu.async_copy(ctx_hbm.at[pl.ds(cb + 256, 64)],
                                     craw.at[2, pl.ds(0, 64)], sem2)]
            for cp in icps:
                cp.wait()
            for v in range(CH // LANES):
                r = traw[0, pl.ds(v * LANES, LANES)]
                tidx[0, pl.ds(v * LANES, LANES)] = jnp.where(
                    r >= SPLIT, r - SPLIT, r)
            for j in range(CIB):
                n = 128 if j < 2 else 64
                for v in range(n // LANES):
                    r = craw[j, pl.ds(v * LANES, LANES)]
                    cidx[j, pl.ds(v * LANES, LANES)] = jnp.where(
                        r >= SPLIT, r - SPLIT, r)
            cps = [pltpu.async_copy(ttab_hbm.at[tidx.at[0]], tgt_sl, sem),
                   pltpu.async_copy(ctab_hbm.at[cidx.at[0]],
                                    ctx_sl.at[pl.ds(0, 128)], sem),
                   pltpu.async_copy(ctab_hbm.at[cidx.at[1]],
                                    ctx_sl.at[pl.ds(128, 128)], sem),
                   pltpu.async_copy(ctab_hbm.at[cidx.at[2, pl.ds(0, 64)]],
                                    ctx_sl.at[pl.ds(256, 64)], sem)]
            for cp in cps:
                cp.wait()

            @pl.loop(0, NG)
            def _grp(g):
                trow = g * LANES + lane
                tr = plsc.load_gather(traw.at[0], [trow])
                tc0 = jnp.where(tr >= SPLIT, DIM, 0)
                pvecs, cc0, accs = [], [], []
                for c in range(NCTX):
                    p = trow * NCTX + c
                    cr = plsc.load_gather(craw, [p >> 7, p & 127])
                    pvecs.append(p)
                    cc0.append(jnp.where(cr >= SPLIT, DIM, 0))
                    accs.append(jnp.zeros((LANES,), jnp.float32))
                for d in range(DIM):
                    tv = plsc.load_gather(tgt_sl, [trow, tc0 + d])
                    for c in range(NCTX):
                        cv = plsc.load_gather(ctx_sl, [pvecs[c], cc0[c] + d])
                        accs[c] = accs[c] + cv * tv
                for c in range(NCTX):
                    plsc.store_scatter(out_v, [pvecs[c]], accs[c])

            pltpu.sync_copy(out_v, out_hbm.at[pl.ds(cb, CH * NCTX)])

    return body


_sc_kernel = _make_kernel()


def kernel(target, context, target_table, context_table):
    tgt1 = target.reshape(B).astype(jnp.int32)
    ctx1 = context.reshape(B * NCTX).astype(jnp.int32)
    ttab = _tc_pack(target_table)
    ctab = _tc_pack(context_table)
    flat = _sc_kernel(tgt1, ctx1, ttab, ctab)
    return flat.reshape(B, NCTX)


# speedup vs baseline: 7.4832x; 1.1125x over previous
import functools

import jax
import jax.numpy as jnp
from jax import lax
from jax.experimental import pallas as pl
from jax.experimental.pallas import tpu as pltpu
from jax.experimental.pallas import tpu_sc as plsc

B = 16384
DIM = 64
NCTX = 5
NC = 2
NS = 16
NW = NC * NS
BPW = B // NW
CH = 64
NCHUNK = BPW // CH
LANES = 16
NG = CH // LANES
CIB = 3
SPLIT = 499712
PH = 500288


def _tc_pack(table):
    t_t = table.T
    eye = jnp.eye(DIM, dtype=jnp.float32)
    bn = 8192

    def body(a_ref, b_ref, eye_ref, o_ref):
        e = eye_ref[...]
        dn = (((0,), (0,)), ((), ()))
        o_ref[:, 0:DIM] = lax.dot_general(
            a_ref[...], e, dn, preferred_element_type=jnp.float32)
        o_ref[:, DIM:128] = lax.dot_general(
            b_ref[...], e, dn, preferred_element_type=jnp.float32)

    nblk = (PH + bn - 1) // bn
    return pl.pallas_call(
        body,
        grid=(nblk,),
        in_specs=[
            pl.BlockSpec((DIM, bn), lambda i: (0, i)),
            pl.BlockSpec((DIM, bn), lambda i: (0, SPLIT // bn + i)),
            pl.BlockSpec((DIM, DIM), lambda i: (0, 0)),
        ],
        out_specs=pl.BlockSpec((bn, 128), lambda i: (i, 0)),
        out_shape=jax.ShapeDtypeStruct((PH, 128), jnp.float32),
    )(t_t, t_t, eye)


def _make_kernel():
    mesh = plsc.VectorSubcoreMesh(core_axis_name="c", subcore_axis_name="s")

    @functools.partial(
        pl.kernel,
        out_type=jax.ShapeDtypeStruct((B * NCTX,), jnp.float32),
        mesh=mesh,
        compiler_params=pltpu.CompilerParams(needs_layout_passes=False),
        scratch_types=[
            pltpu.VMEM((1, CH), jnp.int32),
            pltpu.VMEM((CIB, 128), jnp.int32),
            pltpu.VMEM((1, CH), jnp.int32),
            pltpu.VMEM((CIB, 128), jnp.int32),
            pltpu.VMEM((CH, 128), jnp.float32),
            pltpu.VMEM((CH * NCTX, 128), jnp.float32),
            pltpu.VMEM((CH * NCTX,), jnp.float32),
            pltpu.SemaphoreType.DMA,
            pltpu.SemaphoreType.DMA,
        ],
    )
    def body(tgt_hbm, ctx_hbm, ttab_hbm, ctab_hbm, out_hbm,
             traw, craw, tidx, cidx, tgt_sl, ctx_sl, out_v, sem, sem2):
        wid = lax.axis_index("s") * NC + lax.axis_index("c")
        lane = lax.iota(jnp.int32, LANES)

        @pl.loop(0, NCHUNK)
        def _chunk(ch):
            base = (wid * NCHUNK + ch) * CH
            cb = base * NCTX
            icps = [pltpu.async_copy(tgt_hbm.at[pl.ds(base, CH)],
                                     traw.at[0], sem2),
                    pltpu.async_copy(ctx_hbm.at[pl.ds(cb, 128)],
                                     craw.at[0], sem2),
                    pltpu.async_copy(ctx_hbm.at[pl.ds(cb + 128, 128)],
                                     craw.at[1], sem2),
                    pltpu.async_copy(ctx_hbm.at[pl.ds(cb + 256, 64)],
                                     craw.at[2, pl.ds(0, 64)], sem2)]
            for cp in icps:
                cp.wait()
            for v in range(CH // LANES):
                r = traw[0, pl.ds(v * LANES, LANES)]
                tidx[0, pl.ds(v * LANES, LANES)] = jnp.where(
                    r >= SPLIT, r - SPLIT, r)
            for j in range(CIB):
                n = 128 if j < 2 else 64
                for v in range(n // LANES):
                    r = craw[j, pl.ds(v * LANES, LANES)]
                    cidx[j, pl.ds(v * LANES, LANES)] = jnp.where(
                        r >= SPLIT, r - SPLIT, r)
            cps = [pltpu.async_copy(ttab_hbm.at[tidx.at[0]], tgt_sl, sem),
                   pltpu.async_copy(ctab_hbm.at[cidx.at[0]],
                                    ctx_sl.at[pl.ds(0, 128)], sem),
                   pltpu.async_copy(ctab_hbm.at[cidx.at[1]],
                                    ctx_sl.at[pl.ds(128, 128)], sem),
                   pltpu.async_copy(ctab_hbm.at[cidx.at[2, pl.ds(0, 64)]],
                                    ctx_sl.at[pl.ds(256, 64)], sem)]
            for cp in cps:
                cp.wait()

            @pl.loop(0, NG)
            def _grp(g):
                trow = g * LANES + lane
                tr = plsc.load_gather(traw.at[0], [trow])
                tc0 = jnp.where(tr >= SPLIT, DIM, 0)
                pvecs, cc0, accs = [], [], []
                for c in range(NCTX):
                    p = trow * NCTX + c
                    cr = plsc.load_gather(craw, [p >> 7, p & 127])
                    pvecs.append(p)
                    cc0.append(jnp.where(cr >= SPLIT, DIM, 0))
                    accs.append(jnp.zeros((LANES,), jnp.float32))
                for d in range(DIM):
                    tv = plsc.load_gather(tgt_sl, [trow, tc0 + d])
                    for c in range(NCTX):
                        cv = plsc.load_gather(ctx_sl, [pvecs[c], cc0[c] + d])
                        accs[c] = accs[c] + cv * tv
                for c in range(NCTX):
                    plsc.store_scatter(out_v, [pvecs[c]], accs[c])

            pltpu.sync_copy(out_v, out_hbm.at[pl.ds(cb, CH * NCTX)])

    return body


_sc_kernel = _make_kernel()


def kernel(target, context, target_table, context_table):
    tgt1 = target.reshape(B).astype(jnp.int32)
    ctx1 = context.reshape(B * NCTX).astype(jnp.int32)
    ttab = _tc_pack(target_table)
    ctab = _tc_pack(context_table)
    flat = _sc_kernel(tgt1, ctx1, ttab, ctab)
    return flat.reshape(B, NCTX)
